# Initial kernel scaffold; baseline (speedup 1.0000x reference)
import jax
import jax.numpy as jnp
from jax.experimental import pallas as pl


def kernel(embed_id, weight):
    raise NotImplementedError("write your pallas kernel here")


# speedup vs baseline: 5.3102x; 5.3102x over previous
import functools

import jax
import jax.numpy as jnp
from jax import lax
from jax.experimental import pallas as pl
from jax.experimental.pallas import tpu as pltpu
from jax.experimental.pallas import tpu_sc as plsc

_info = plsc.get_sparse_core_info()
_NC, _NS = _info.num_cores, _info.num_subcores
_NW = _NC * _NS

_G = 128
_GPC = 16
_CHUNK = _G * _GPC


def _sc_gather(idx, weight, n_per_w, d):
    n_chunks = n_per_w // _CHUNK
    n_groups = n_per_w // _G

    mesh = plsc.VectorSubcoreMesh(core_axis_name="c", subcore_axis_name="s")

    @functools.partial(
        pl.kernel,
        out_type=jax.ShapeDtypeStruct((_NW * n_per_w, d), jnp.float32),
        mesh=mesh,
        compiler_params=pltpu.CompilerParams(use_tc_tiling_on_sc=False),
        scratch_types=[
            pltpu.VMEM((n_groups, _G), jnp.int32),
            pltpu.VMEM((_CHUNK, d), jnp.float32),
            pltpu.SemaphoreType.DMA,
        ],
    )
    def body(idx_hbm, table_hbm, out_hbm, idx_v, rows_v, sem):
        wid = lax.axis_index("s") * _NC + lax.axis_index("c")
        base = wid * n_per_w
        pltpu.sync_copy(idx_hbm.at[wid], idx_v)

        def chunk_body(c, carry):
            g0 = c * _GPC
            copies = [
                pltpu.async_copy(
                    table_hbm.at[idx_v.at[g0 + g]],
                    rows_v.at[pl.ds(g * _G, _G)],
                    sem,
                )
                for g in range(_GPC)
            ]
            for cp in copies:
                cp.wait()
            pltpu.sync_copy(
                rows_v, out_hbm.at[pl.ds(base + c * _CHUNK, _CHUNK)]
            )
            return carry

        lax.fori_loop(0, n_chunks, chunk_body, 0)

    return body(idx, weight)


def kernel(embed_id, weight):
    b0, b1 = embed_id.shape
    v, d = weight.shape
    n = b0 * b1
    n_per_w = n // _NW
    idx = embed_id.astype(jnp.int32).reshape(_NW, n_per_w // _G, _G)
    out = _sc_gather(idx, weight, n_per_w, d)
    return out.reshape(b0, b1, d)


# speedup vs baseline: 5.3179x; 1.0015x over previous
import functools

import jax
import jax.numpy as jnp
from jax import lax
from jax.experimental import pallas as pl
from jax.experimental.pallas import tpu as pltpu
from jax.experimental.pallas import tpu_sc as plsc

_info = plsc.get_sparse_core_info()
_NC, _NS = _info.num_cores, _info.num_subcores
_NW = _NC * _NS

_G = 128
_GPC = 8
_CHUNK = _G * _GPC


def _sc_gather(idx, weight, n_per_w, d):
    n_chunks = n_per_w // _CHUNK
    n_pairs = n_chunks // 2
    n_groups = n_per_w // _G

    mesh = plsc.VectorSubcoreMesh(core_axis_name="c", subcore_axis_name="s")

    @functools.partial(
        pl.kernel,
        out_type=jax.ShapeDtypeStruct((_NW * n_per_w, d), jnp.float32),
        mesh=mesh,
        compiler_params=pltpu.CompilerParams(use_tc_tiling_on_sc=False),
        scratch_types=[
            pltpu.VMEM((n_groups, _G), jnp.int32),
            pltpu.VMEM((2, _CHUNK, d), jnp.float32),
            pltpu.SemaphoreType.DMA,
            pltpu.SemaphoreType.DMA,
            pltpu.SemaphoreType.DMA,
            pltpu.SemaphoreType.DMA,
        ],
    )
    def body(idx_hbm, table_hbm, out_hbm, idx_v, rows_v,
             gsem0, gsem1, wsem0, wsem1):
        wid = lax.axis_index("s") * _NC + lax.axis_index("c")
        base = wid * n_per_w
        pltpu.sync_copy(idx_hbm.at[wid], idx_v)
        gsems = (gsem0, gsem1)
        wsems = (wsem0, wsem1)

        def fire_gathers(c, b):
            g0 = c * _GPC
            for g in range(_GPC):
                pltpu.async_copy(
                    table_hbm.at[idx_v.at[g0 + g]],
                    rows_v.at[b].at[pl.ds(g * _G, _G)],
                    gsems[b],
                )

        def drain_gathers(b):
            pltpu.make_async_copy(
                out_hbm.at[pl.ds(base, _CHUNK)], rows_v.at[b], gsems[b]
            ).wait()

        def fire_write(c, b):
            pltpu.async_copy(
                rows_v.at[b],
                out_hbm.at[pl.ds(base + c * _CHUNK, _CHUNK)],
                wsems[b],
            )

        def wait_write(b):
            pltpu.make_async_copy(
                rows_v.at[b], out_hbm.at[pl.ds(base, _CHUNK)], wsems[b]
            ).wait()

        fire_gathers(0, 0)

        def pair_body(i, carry):
            @pl.when(i > 0)
            def _():
                wait_write(1)
            fire_gathers(2 * i + 1, 1)
            drain_gathers(0)
            fire_write(2 * i, 0)

            @pl.when(i < n_pairs - 1)
            def _():
                wait_write(0)
                fire_gathers(2 * i + 2, 0)
            drain_gathers(1)
            fire_write(2 * i + 1, 1)
            return carry

        lax.fori_loop(0, n_pairs, pair_body, 0)
        wait_write(0)
        wait_write(1)

    return body(idx, weight)


def kernel(embed_id, weight):
    b0, b1 = embed_id.shape
    v, d = weight.shape
    n = b0 * b1
    n_per_w = n // _NW
    idx = embed_id.astype(jnp.int32).reshape(_NW, n_per_w // _G, _G)
    out = _sc_gather(idx, weight, n_per_w, d)
    return out.reshape(b0, b1, d)


# speedup vs baseline: 13.7618x; 2.5878x over previous
import functools

import jax
import jax.numpy as jnp
from jax import lax
from jax.experimental import pallas as pl
from jax.experimental.pallas import tpu as pltpu
from jax.experimental.pallas import tpu_sc as plsc

_info = plsc.get_sparse_core_info()
_NC, _NS = _info.num_cores, _info.num_subcores
_NW = _NC * _NS
_NPG = 4
_NTG = _NW // _NPG


def _sc_plane_gather(idx, wt):
    mesh = plsc.VectorSubcoreMesh(core_axis_name="c", subcore_axis_name="s")

    @functools.partial(
        pl.kernel,
        out_type=jax.ShapeDtypeStruct((256, 4, 8, 8, 128), jnp.float32),
        mesh=mesh,
        compiler_params=pltpu.CompilerParams(
            use_tc_tiling_on_sc=False, needs_layout_passes=False
        ),
        scratch_types=[
            pltpu.VMEM((8 * 8192,), jnp.float32),
            pltpu.VMEM((4, 8, 8, 128), jnp.int32),
            pltpu.VMEM((2, 8, 8, 128), jnp.float32),
            pltpu.SemaphoreType.DMA,
            pltpu.SemaphoreType.DMA,
        ],
    )
    def body(idx_hbm, wt_flat_hbm, out_hbm, planes_v, idx_v, stage_v,
             wsem0, wsem1):
        wid = lax.axis_index("s") * _NC + lax.axis_index("c")
        tg = wid // _NPG
        pg = wid % _NPG
        pltpu.sync_copy(wt_flat_hbm.at[pl.ds(8 * 8192 * pg, 8 * 8192)], planes_v)
        pltpu.sync_copy(idx_hbm.at[pl.ds(4 * tg, 4)], idx_v)
        wsems = (wsem0, wsem1)

        def wait_write(slot):
            pltpu.make_async_copy(
                stage_v.at[slot], out_hbm.at[0].at[pg], wsems[slot]
            ).wait()

        def pair_body(i, carry):
            for slot in range(2):
                @pl.when(i > 0)
                def _():
                    wait_write(slot)
                bl = 2 * i + slot
                rt = lax.shift_right_logical(bl, 3)
                sb = lax.bitwise_and(bl, 7)

                def ct_body(ct, c2):
                    for l16 in range(8):
                        vidx = idx_v[rt, ct, sb, pl.ds(16 * l16, 16)]
                        vals = [
                            plsc.load_gather(planes_v, [vidx + (8192 * s)])
                            for s in range(8)
                        ]
                        for s in range(8):
                            stage_v[slot, ct, s, pl.ds(16 * l16, 16)] = vals[s]
                    return c2

                lax.fori_loop(0, 8, ct_body, 0)
                b = 32 * tg + bl
                pltpu.async_copy(
                    stage_v.at[slot], out_hbm.at[b].at[pg], wsems[slot]
                )
            return carry

        lax.fori_loop(0, 16, pair_body, 0)
        wait_write(0)
        wait_write(1)

    return body(idx, wt)


def kernel(embed_id, weight):
    idx = embed_id.astype(jnp.int32).reshape(32, 8, 8, 128).transpose(0, 2, 1, 3)
    wt = jnp.transpose(weight).reshape(-1)
    out5 = _sc_plane_gather(idx, wt)
    out = out5.transpose(0, 1, 3, 2, 4).reshape(256, 32, 1024).transpose(0, 2, 1)
    return out


# speedup vs baseline: 18.9770x; 1.3790x over previous
import functools

import jax
import jax.numpy as jnp
from jax import lax
from jax.experimental import pallas as pl
from jax.experimental.pallas import tpu as pltpu
from jax.experimental.pallas import tpu_sc as plsc

_info = plsc.get_sparse_core_info()
_NC, _NS = _info.num_cores, _info.num_subcores
_NW = _NC * _NS
_NPG = 4
_NTG = _NW // _NPG


def _sc_plane_gather(idx, wt):
    mesh = plsc.VectorSubcoreMesh(core_axis_name="c", subcore_axis_name="s")

    @functools.partial(
        pl.kernel,
        out_type=jax.ShapeDtypeStruct((256, 4, 8, 8, 128), jnp.float32),
        mesh=mesh,
        compiler_params=pltpu.CompilerParams(
            use_tc_tiling_on_sc=False, needs_layout_passes=False
        ),
        scratch_types=[
            pltpu.VMEM((8 * 8192,), jnp.float32),
            pltpu.VMEM((4, 8, 8, 128), jnp.int32),
            pltpu.VMEM((2, 8, 8, 128), jnp.float32),
            pltpu.SemaphoreType.DMA,
            pltpu.SemaphoreType.DMA,
        ],
    )
    def body(idx_hbm, wt_flat_hbm, out_hbm, planes_v, idx_v, stage_v,
             wsem0, wsem1):
        wid = lax.axis_index("s") * _NC + lax.axis_index("c")
        tg = wid // _NPG
        pg = wid % _NPG
        pltpu.sync_copy(wt_flat_hbm.at[pl.ds(8 * 8192 * pg, 8 * 8192)], planes_v)
        pltpu.sync_copy(idx_hbm.at[pl.ds(4 * tg, 4)], idx_v)
        wsems = (wsem0, wsem1)

        def wait_write(slot):
            pltpu.make_async_copy(
                stage_v.at[slot], out_hbm.at[0].at[pg], wsems[slot]
            ).wait()

        def pair_body(i, carry):
            for slot in range(2):
                @pl.when(i > 0)
                def _():
                    wait_write(slot)
                bl = 2 * i + slot
                rt = lax.shift_right_logical(bl, 3)
                sb = lax.bitwise_and(bl, 7)

                def ct_body(ct, c2):
                    vidxs = [
                        idx_v[rt, ct, sb, pl.ds(16 * l16, 16)]
                        for l16 in range(8)
                    ]

                    def gather(s, l16):
                        return plsc.load_gather(
                            planes_v.at[pl.ds(8192 * s, 8192)], [vidxs[l16]]
                        )

                    prev = None
                    for l16 in range(8):
                        vals = []
                        for s in range(8):
                            vals.append(gather(s, l16))
                            if prev is not None:
                                stage_v[slot, ct, s, pl.ds(16 * (l16 - 1), 16)] = prev[s]
                        prev = vals
                    for s in range(8):
                        stage_v[slot, ct, s, pl.ds(16 * 7, 16)] = prev[s]
                    return c2

                lax.fori_loop(0, 8, ct_body, 0)
                b = 32 * tg + bl
                pltpu.async_copy(
                    stage_v.at[slot], out_hbm.at[b].at[pg], wsems[slot]
                )
            return carry

        lax.fori_loop(0, 16, pair_body, 0)
        wait_write(0)
        wait_write(1)

    return body(idx, wt)


def kernel(embed_id, weight):
    idx = embed_id.astype(jnp.int32).reshape(32, 8, 8, 128).transpose(0, 2, 1, 3)
    wt = jnp.transpose(weight).reshape(-1)
    out5 = _sc_plane_gather(idx, wt)
    out = out5.transpose(0, 1, 3, 2, 4).reshape(256, 32, 1024).transpose(0, 2, 1)
    return out
